# baseline (speedup 1.0000x reference)
import functools

import jax
import jax.numpy as jnp
from jax import lax
from jax.experimental import pallas as pl
from jax.experimental.pallas import tpu as pltpu
from jax.experimental.pallas import tpu_sc as plsc

VOCAB = 1_000_000
EMB = 16
ROWS = 16384
COLS = 200
B_TOTAL = ROWS * COLS

_info = plsc.get_sparse_core_info()
NUM_CORES = _info.num_cores
NUM_SUBCORES = _info.num_subcores
NW = NUM_CORES * NUM_SUBCORES

JT = COLS // 8
IB_TOTAL = ROWS // 128
IB_PER_W = IB_TOTAL // NW
NB = IB_PER_W * JT
BLK = 1024


def _make_kernel():
    mesh = plsc.VectorSubcoreMesh(core_axis_name="c", subcore_axis_name="s")

    @functools.partial(
        pl.kernel,
        mesh=mesh,
        out_type=jax.ShapeDtypeStruct((COLS, 2, 128, 8, 128), jnp.float32),
        scratch_types=[
            pltpu.VMEM((2, BLK), jnp.int32),
            pltpu.VMEM((2, BLK, EMB), jnp.float32),
            pltpu.VMEM((2, 8, EMB, 128), jnp.float32),
        ] + [pltpu.SemaphoreType.DMA] * 6,
        compiler_params=pltpu.CompilerParams(
            use_tc_tiling_on_sc=False, needs_layout_passes=False),
    )
    def emb_kernel(xt_hbm, table_hbm, out_hbm, idx_v, rows_v, rowst_v,
                   ld0, ld1, g0, g1, st0, st1):
        sem_ld = (ld0, ld1)
        sem_g = (g0, g1)
        sem_st = (st0, st1)
        wid = lax.axis_index("s") * NUM_CORES + lax.axis_index("c")
        iota = lax.iota(jnp.int32, 16)

        def block_coords(n):
            ib = n // JT
            jt = n - ib * JT
            return jt, wid * IB_PER_W + ib

        def fire_ld(n, b):
            jt, tcg = block_coords(n)
            pltpu.async_copy(xt_hbm.at[jt, tcg], idx_v.at[b], sem_ld[b])

        def wait_ld(b):
            pltpu.make_async_copy(xt_hbm.at[0, 0], idx_v.at[b],
                                  sem_ld[b]).wait()

        def fire_gather(b):
            pltpu.async_copy(table_hbm.at[idx_v.at[b]], rows_v.at[b],
                             sem_g[b])

        def wait_gather(b):
            pltpu.make_async_copy(table_hbm.at[idx_v.at[b]], rows_v.at[b],
                                  sem_g[b]).wait()

        def fire_st(n, b, j2, tr):
            jt, tcg = block_coords(n)
            pltpu.async_copy(
                rowst_v.at[b, j2, pl.ds(8 * tr, 8)],
                out_hbm.at[8 * jt + j2, tr, tcg], sem_st[b])

        def wait_st_all(b):
            for _ in range(16):
                pltpu.make_async_copy(
                    rowst_v.at[b, 0, pl.ds(0, 8)],
                    out_hbm.at[0, 0, 0], sem_st[b]).wait()

        def transpose_and_store(n, b):
            for j2 in range(8):
                @pl.loop(0, 128, step=8)
                def _(ii):
                    iota_l = lax.iota(jnp.int32, 16)
                    row_base = jnp.full((16,), j2 * 128 + ii, jnp.int32)
                    col_base = jnp.full((16,), ii, jnp.int32)
                    for u in range(8):
                        row = plsc.load_gather(
                            rows_v.at[b], [row_base + u, iota_l])
                        plsc.store_scatter(
                            rowst_v.at[b, j2], [iota_l, col_base + u], row)

                fire_st(n, b, j2, 0)
                fire_st(n, b, j2, 1)
            wait_st_all(b)

        fire_ld(0, 0)
        fire_ld(1, 1)
        wait_ld(0)
        fire_gather(0)

        def body(n, b):
            wait_gather(b)

            @pl.when(n + 1 < NB)
            def _():
                wait_ld(1 - b)
                fire_gather(1 - b)

            @pl.when(n + 2 < NB)
            def _():
                fire_ld(n + 2, b)

            transpose_and_store(n, b)

        @pl.loop(0, NB, step=2)
        def _(h):
            for b in range(2):
                body(h + b, b)

    return emb_kernel


_emb_kernel = _make_kernel()


def kernel(x, table):
    xt = (x.transpose(1, 0).reshape(JT, 8, 128, 128)
          .transpose(0, 2, 1, 3).reshape(JT, 128, BLK))
    t = _emb_kernel(xt, table)
    return t.transpose((2, 4, 0, 1, 3)).reshape(ROWS, COLS, EMB)


# speedup vs baseline: 1.6236x; 1.6236x over previous
import functools

import jax
import jax.numpy as jnp
from jax import lax
from jax.experimental import pallas as pl
from jax.experimental.pallas import tpu as pltpu
from jax.experimental.pallas import tpu_sc as plsc

VOCAB = 1_000_000
EMB = 16
ROWS = 16384
COLS = 200
B_TOTAL = ROWS * COLS

_info = plsc.get_sparse_core_info()
NUM_CORES = _info.num_cores
NUM_SUBCORES = _info.num_subcores
NW = NUM_CORES * NUM_SUBCORES

JT = COLS // 8
IB_TOTAL = ROWS // 128
IB_PER_W = IB_TOTAL // NW
NB = IB_PER_W * JT
BLK = 1024


def _make_kernel():
    mesh = plsc.VectorSubcoreMesh(core_axis_name="c", subcore_axis_name="s")

    @functools.partial(
        pl.kernel,
        mesh=mesh,
        out_type=jax.ShapeDtypeStruct((COLS, 2, 128, 8, 128), jnp.float32),
        scratch_types=[
            pltpu.VMEM((2, BLK), jnp.int32),
            pltpu.VMEM((2, BLK, EMB), jnp.float32),
            pltpu.VMEM((2, 8, EMB, 129), jnp.float32),
        ] + [pltpu.SemaphoreType.DMA] * 6,
        compiler_params=pltpu.CompilerParams(
            use_tc_tiling_on_sc=False, needs_layout_passes=False),
    )
    def emb_kernel(xt_hbm, table_hbm, out_hbm, idx_v, rows_v, rowst_v,
                   ld0, ld1, g0, g1, st0, st1):
        sem_ld = (ld0, ld1)
        sem_g = (g0, g1)
        sem_st = (st0, st1)
        wid = lax.axis_index("s") * NUM_CORES + lax.axis_index("c")
        iota = lax.iota(jnp.int32, 16)

        def block_coords(n):
            ib = n // JT
            jt = n - ib * JT
            return jt, wid * IB_PER_W + ib

        def fire_ld(n, b):
            jt, tcg = block_coords(n)
            pltpu.async_copy(xt_hbm.at[jt, tcg], idx_v.at[b], sem_ld[b])

        def wait_ld(b):
            pltpu.make_async_copy(xt_hbm.at[0, 0], idx_v.at[b],
                                  sem_ld[b]).wait()

        def fire_gather(b):
            pltpu.async_copy(table_hbm.at[idx_v.at[b]], rows_v.at[b],
                             sem_g[b])

        def wait_gather(b):
            pltpu.make_async_copy(table_hbm.at[idx_v.at[b]], rows_v.at[b],
                                  sem_g[b]).wait()

        def fire_st(n, b, j2, tr):
            jt, tcg = block_coords(n)
            pltpu.async_copy(
                rowst_v.at[b, j2, pl.ds(8 * tr, 8), pl.ds(0, 128)],
                out_hbm.at[8 * jt + j2, tr, tcg], sem_st[b])

        def wait_st_all(b):
            for _ in range(16):
                pltpu.make_async_copy(
                    rowst_v.at[b, 0, pl.ds(0, 8), pl.ds(0, 128)],
                    out_hbm.at[0, 0, 0], sem_st[b]).wait()

        def transpose_and_store(n, b):
            for j2 in range(8):
                @pl.loop(0, 128, step=8)
                def _(ii):
                    iota_l = lax.iota(jnp.int32, 16)
                    row_base = jnp.full((16,), j2 * 128 + ii, jnp.int32)
                    col_base = jnp.full((16,), ii, jnp.int32)
                    for u in range(8):
                        row = plsc.load_gather(
                            rows_v.at[b], [row_base + u, iota_l])
                        plsc.store_scatter(
                            rowst_v.at[b, j2], [iota_l, col_base + u], row)

                fire_st(n, b, j2, 0)
                fire_st(n, b, j2, 1)

        fire_ld(0, 0)
        fire_ld(1, 1)
        wait_ld(0)
        fire_gather(0)

        def body(n, b, drain):
            wait_gather(b)

            @pl.when(n + 1 < NB)
            def _():
                wait_ld(1 - b)
                fire_gather(1 - b)

            @pl.when(n + 2 < NB)
            def _():
                fire_ld(n + 2, b)

            if drain:
                wait_st_all(b)
            transpose_and_store(n, b)

        body(0, 0, False)
        body(1, 1, False)

        @pl.loop(2, NB, step=2)
        def _(h):
            for b in range(2):
                body(h + b, b, True)

        wait_st_all(0)
        wait_st_all(1)

    return emb_kernel


_emb_kernel = _make_kernel()


def kernel(x, table):
    xt = (x.transpose(1, 0).reshape(JT, 8, 128, 128)
          .transpose(0, 2, 1, 3).reshape(JT, 128, BLK))
    t = _emb_kernel(xt, table)
    return t.transpose((2, 4, 0, 1, 3)).reshape(ROWS, COLS, EMB)
